# Initial kernel scaffold; baseline (speedup 1.0000x reference)
import jax
import jax.numpy as jnp
from jax.experimental import pallas as pl


def kernel(variables, ground, lits, clause_ids, clause_var, W_neg, b_neg, W_vc, b_vc, W_cc, b_cc, W_gc, b_gc, Wz, Uz, bz, Wr, Ur, br, Wh, Uh, bh):
    raise NotImplementedError("write your pallas kernel here")


# speedup vs baseline: 2.0780x; 2.0780x over previous
import functools

import jax
import jax.numpy as jnp
from jax import lax
from jax.experimental import pallas as pl
from jax.experimental.pallas import tpu as pltpu
from jax.experimental.pallas import tpu_sc as plsc

_NC, _NS = 2, 16
_K = 128
_ZR = 128


def _dot_t(x, w):
    return lax.dot_general(x, w, (((1,), (1,)), ((), ())),
                           preferred_element_type=jnp.float32)


def _tc_table(variables, w_neg, b_neg):
    n, d = variables.shape
    br = 1000
    grid = n // br

    def body(v_ref, w_ref, b_ref, o_ref):
        v = v_ref[...]
        o_ref[0] = v
        o_ref[1] = _dot_t(v, w_ref[...]) + b_ref[...]

    out = pl.pallas_call(
        body,
        grid=(grid,),
        in_specs=[
            pl.BlockSpec((br, d), lambda i: (i, 0)),
            pl.BlockSpec((d, d), lambda i: (0, 0)),
            pl.BlockSpec((1, d), lambda i: (0, 0)),
        ],
        out_specs=pl.BlockSpec((2, br, d), lambda i: (0, i, 0)),
        out_shape=jax.ShapeDtypeStruct((2, n, d), jnp.float32),
    )(variables, w_neg, b_neg.reshape(1, d))
    return out.reshape(2 * n, d)


def _tc_clause(s_sum, w_vc, b_vc):
    c, d = s_sum.shape
    br = 2016
    grid = c // br

    def body(s_ref, w_ref, b_ref, o_ref):
        o_ref[...] = jnp.tanh(_dot_t(s_ref[...], w_ref[...]) + b_ref[...])

    return pl.pallas_call(
        body,
        grid=(grid,),
        in_specs=[
            pl.BlockSpec((br, d), lambda i: (i, 0)),
            pl.BlockSpec((d, d), lambda i: (0, 0)),
            pl.BlockSpec((1, d), lambda i: (0, 0)),
        ],
        out_specs=pl.BlockSpec((br, d), lambda i: (i, 0)),
        out_shape=jax.ShapeDtypeStruct((c, d), jnp.float32),
    )(s_sum, w_vc, b_vc.reshape(1, d))


def _tc_final(variables, ground, vs, cnt,
              w_cc, b_cc, w_g1, w_g2, b_gc,
              wz, uz, bz, wr, ur, br_, wh, uh, bh):
    n, d = variables.shape
    g = ground.shape[1]
    br = 2000
    grid = n // br

    def body(v_ref, g_ref, vs_ref, cnt_ref,
             wcc_ref, bcc_ref, wg1_ref, wg2_ref, bgc_ref,
             wz_ref, uz_ref, bz_ref, wr_ref, ur_ref, brr_ref,
             wh_ref, uh_ref, bh_ref, o_ref):
        v = v_ref[...]
        vsum = vs_ref[...]
        counts = cnt_ref[...]
        combined = jnp.tanh(_dot_t(vsum, wcc_ref[...]) + bcc_ref[...])
        new_emb = jnp.tanh(_dot_t(g_ref[...], wg1_ref[...])
                           + _dot_t(combined, wg2_ref[...]) + bgc_ref[...])
        has = counts[:, :1] > 0.0
        av = jnp.where(has, new_emb, v)
        z = jax.nn.sigmoid(_dot_t(av, wz_ref[...]) + _dot_t(v, uz_ref[...])
                           + bz_ref[...])
        r = jax.nn.sigmoid(_dot_t(av, wr_ref[...]) + _dot_t(v, ur_ref[...])
                           + brr_ref[...])
        h_t = jnp.tanh(_dot_t(av, wh_ref[...]) + _dot_t(r * v, uh_ref[...])
                       + bh_ref[...])
        o_ref[...] = (1.0 - z) * v + z * h_t

    def full(shape):
        return pl.BlockSpec(shape, lambda i: tuple(0 for _ in shape))

    return pl.pallas_call(
        body,
        grid=(grid,),
        in_specs=[
            pl.BlockSpec((br, d), lambda i: (i, 0)),
            pl.BlockSpec((br, g), lambda i: (i, 0)),
            pl.BlockSpec((br, d), lambda i: (i, 0)),
            pl.BlockSpec((br, d), lambda i: (i, 0)),
            full((d, d)), full((1, d)),
            full((d, g)), full((d, d)), full((1, d)),
            full((d, d)), full((d, d)), full((1, d)),
            full((d, d)), full((d, d)), full((1, d)),
            full((d, d)), full((d, d)), full((1, d)),
        ],
        out_specs=pl.BlockSpec((br, d), lambda i: (i, 0)),
        out_shape=jax.ShapeDtypeStruct((n, d), jnp.float32),
    )(variables, ground, vs, cnt,
      w_cc, b_cc.reshape(1, d), w_g1, w_g2, b_gc.reshape(1, d),
      wz, uz, bz.reshape(1, d), wr, ur, br_.reshape(1, d),
      wh, uh, bh.reshape(1, d))


def _sc_clause_sum(table, lits_p, cids_p, blk_off, zeros_hbm, c, d, cb, nblk):
    blk_per_sc = nblk // _NC
    stripe = cb // _NS
    zchunks = []
    off = 0
    while off < stripe:
        step = min(_ZR, stripe - off)
        zchunks.append((off, step))
        off += step
    assert stripe % 8 == 0
    groups = _K // 16

    mesh = plsc.VectorSubcoreMesh(core_axis_name="c", subcore_axis_name="s")

    @functools.partial(
        pl.kernel,
        out_type=jax.ShapeDtypeStruct((c, d), jnp.float32),
        mesh=mesh,
        scratch_types=[
            pltpu.VMEM((_K,), jnp.int32),
            pltpu.VMEM((_K,), jnp.int32),
            pltpu.VMEM((_K, d), jnp.float32),
            pltpu.VMEM((_ZR, d), jnp.float32),
            pltpu.VMEM_SHARED((cb + 8, d), jnp.float32),
            pltpu.VMEM((64,), jnp.int32),
            pltpu.SemaphoreType.DMA,
        ],
    )
    def k(table_h, lits_h, cids_h, blk_h, zer_h, out_h,
          lit_v, cid_v, rows_v, zv, acc, blk_v, sem):
        ci = lax.axis_index("c")
        si = lax.axis_index("s")
        pltpu.sync_copy(blk_h, blk_v)
        pltpu.sync_copy(zer_h, zv)

        for b_local in range(blk_per_sc):
            b = ci * blk_per_sc + b_local
            base = b * cb
            bw = blk_v[pl.ds(b, 16)]
            lo = bw[0]
            hi = bw[1]

            for off, step in zchunks:
                pltpu.sync_copy(zv.at[pl.ds(0, step)],
                                acc.at[pl.ds(si * stripe + off, step)])
            plsc.subcore_barrier()

            per = (hi - lo + _NS - 1) // _NS
            lo_t = lo + si * per
            hi_t = jnp.minimum(lo_t + per, hi)
            e0 = (lo_t // 8) * 8
            nch = jnp.maximum((hi_t - e0 + _K - 1) // _K, 0)

            def chunk(kk, carry):
                estart = e0 + kk * _K
                pltpu.sync_copy(lits_h.at[pl.ds(estart, _K)], lit_v)
                pltpu.sync_copy(cids_h.at[pl.ds(estart, _K)], cid_v)
                for j in range(groups):
                    sl = pl.ds(j * 16, 16)
                    pos = estart + j * 16 + lax.iota(jnp.int32, 16)
                    m = (pos >= lo_t) & (pos < hi_t)
                    lit_v[sl] = jnp.where(m, lit_v[sl], 0)
                    cid_v[sl] = jnp.where(m, cid_v[sl] - base, cb)
                pltpu.async_copy(table_h.at[lit_v], rows_v, sem).wait()
                pltpu.sync_copy(rows_v, acc.at[cid_v], add=True)
                return carry

            lax.fori_loop(0, nch, chunk, 0)
            plsc.subcore_barrier()

            pltpu.sync_copy(acc.at[pl.ds(si * stripe, stripe)],
                            out_h.at[pl.ds(base + si * stripe, stripe)])

    return k(table, lits_p, cids_p, blk_off, zeros_hbm)


def _sc_var_sum(clause_emb, clause_var, zeros_hbm, n_pad, c, d):
    nchunks = c // _K
    n_half = n_pad // _NC
    stripe = n_half // _NS
    zchunks = []
    off = 0
    while off < stripe:
        step = min(_ZR, stripe - off)
        zchunks.append((off, step))
        off += step
    assert stripe % 8 == 0 and c % _K == 0 and n_pad % (_NC * _NS * 8) == 0
    groups = _K // 16

    mesh = plsc.VectorSubcoreMesh(core_axis_name="c", subcore_axis_name="s")

    @functools.partial(
        pl.kernel,
        out_type=jax.ShapeDtypeStruct((n_pad, d), jnp.float32),
        mesh=mesh,
        scratch_types=[
            pltpu.VMEM((_K,), jnp.int32),
            pltpu.VMEM((_K, d), jnp.float32),
            pltpu.VMEM((_ZR, d), jnp.float32),
            pltpu.VMEM_SHARED((n_half + 8, d), jnp.float32),
        ],
    )
    def k(cemb_h, cvar_h, zer_h, vs_h, idx_v, rows_v, zv, vsacc):
        ci = lax.axis_index("c")
        si = lax.axis_index("s")
        vbase = ci * n_half
        pltpu.sync_copy(zer_h, zv)

        for off, step in zchunks:
            pltpu.sync_copy(zv.at[pl.ds(0, step)],
                            vsacc.at[pl.ds(si * stripe + off, step)])
        plsc.subcore_barrier()

        nch = (nchunks - si + _NS - 1) // _NS

        def chunk(kk, carry):
            ch = si + kk * _NS
            pltpu.sync_copy(cvar_h.at[pl.ds(ch * _K, _K)], idx_v)
            pltpu.sync_copy(cemb_h.at[pl.ds(ch * _K, _K)], rows_v)
            for j in range(groups):
                sl = pl.ds(j * 16, 16)
                rel = idx_v[sl] - vbase
                m = (rel >= 0) & (rel < n_half)
                idx_v[sl] = jnp.where(m, rel, n_half)
            pltpu.sync_copy(rows_v, vsacc.at[idx_v], add=True)
            return carry

        lax.fori_loop(0, nch, chunk, 0)
        plsc.subcore_barrier()

        pltpu.sync_copy(vsacc.at[pl.ds(si * stripe, stripe)],
                        vs_h.at[pl.ds(vbase + si * stripe, stripe)])

    return k(clause_emb, clause_var, zeros_hbm)


def _sc_var_count(clause_var, zeros_hbm, ones_hbm, n_pad, c, d):
    nchunks = c // _K
    n_half = n_pad // _NC
    stripe = n_half // _NS
    zchunks = []
    off = 0
    while off < stripe:
        step = min(_ZR, stripe - off)
        zchunks.append((off, step))
        off += step
    groups = _K // 16

    mesh = plsc.VectorSubcoreMesh(core_axis_name="c", subcore_axis_name="s")

    @functools.partial(
        pl.kernel,
        out_type=jax.ShapeDtypeStruct((n_pad, d), jnp.float32),
        mesh=mesh,
        scratch_types=[
            pltpu.VMEM((_K,), jnp.int32),
            pltpu.VMEM((_K, d), jnp.float32),
            pltpu.VMEM((_ZR, d), jnp.float32),
            pltpu.VMEM_SHARED((n_half + 8, d), jnp.float32),
        ],
    )
    def k(cvar_h, zer_h, ones_h, cnt_h, idx_v, ones_v, zv, cntacc):
        ci = lax.axis_index("c")
        si = lax.axis_index("s")
        vbase = ci * n_half
        pltpu.sync_copy(zer_h, zv)
        pltpu.sync_copy(ones_h, ones_v)

        for off, step in zchunks:
            pltpu.sync_copy(zv.at[pl.ds(0, step)],
                            cntacc.at[pl.ds(si * stripe + off, step)])
        plsc.subcore_barrier()

        nch = (nchunks - si + _NS - 1) // _NS

        def chunk(kk, carry):
            ch = si + kk * _NS
            pltpu.sync_copy(cvar_h.at[pl.ds(ch * _K, _K)], idx_v)
            for j in range(groups):
                sl = pl.ds(j * 16, 16)
                rel = idx_v[sl] - vbase
                m = (rel >= 0) & (rel < n_half)
                idx_v[sl] = jnp.where(m, rel, n_half)
            pltpu.sync_copy(ones_v, cntacc.at[idx_v], add=True)
            return carry

        lax.fori_loop(0, nch, chunk, 0)
        plsc.subcore_barrier()

        pltpu.sync_copy(cntacc.at[pl.ds(si * stripe, stripe)],
                        cnt_h.at[pl.ds(vbase + si * stripe, stripe)])

    return k(clause_var, zeros_hbm, ones_hbm)


def kernel(variables, ground, lits, clause_ids, clause_var,
           W_neg, b_neg, W_vc, b_vc, W_cc, b_cc, W_gc, b_gc,
           Wz, Uz, bz, Wr, Ur, br, Wh, Uh, bh):
    n, d = variables.shape
    g = ground.shape[1]
    e = lits.shape[0]
    c = clause_var.shape[0]

    cb = 1920
    nblk = -(-c // cb)
    nblk += nblk % _NC
    c_pad = nblk * cb
    n_pad = -(-n // (_NC * _NS * 8)) * (_NC * _NS * 8)

    lits32 = lits.astype(jnp.int32)
    cids32 = clause_ids.astype(jnp.int32)
    cvar32 = clause_var.astype(jnp.int32)

    pad = jnp.zeros((_K,), jnp.int32)
    lits_p = jnp.concatenate([lits32, pad])
    cids_p = jnp.concatenate([cids32, pad])

    bounds = jnp.arange(nblk + 1, dtype=jnp.int32) * cb
    blk_off = jnp.searchsorted(cids32, bounds).astype(jnp.int32)
    blk_off = jnp.concatenate(
        [blk_off, jnp.zeros((64 - nblk - 1,), jnp.int32)])

    zeros_hbm = jnp.zeros((_ZR, d), jnp.float32)
    ones_hbm = jnp.ones((_K, d), jnp.float32)

    table = _tc_table(variables, W_neg, b_neg)
    s_sum = _sc_clause_sum(table, lits_p, cids_p, blk_off, zeros_hbm,
                           c_pad, d, cb, nblk)
    clause_emb = _tc_clause(s_sum, W_vc, b_vc)
    vs = _sc_var_sum(clause_emb, cvar32, zeros_hbm, n_pad, c, d)
    cnt = _sc_var_count(cvar32, zeros_hbm, ones_hbm, n_pad, c, d)
    vs = vs[:n]
    cnt = cnt[:n]
    return _tc_final(variables, ground, vs, cnt,
                     W_cc, b_cc, W_gc[:, :g], W_gc[:, g:], b_gc,
                     Wz, Uz, bz, Wr, Ur, br, Wh, Uh, bh)
